# Initial kernel scaffold; baseline (speedup 1.0000x reference)
import jax
import jax.numpy as jnp
from jax.experimental import pallas as pl


def kernel(inputs, labels, embed_table, W_out, b_out):
    raise NotImplementedError("write your pallas kernel here")


# speedup vs baseline: 1.8872x; 1.8872x over previous
import jax
import jax.numpy as jnp
from jax import lax
from jax.experimental import pallas as pl
from jax.experimental.pallas import tpu as pltpu
from jax.experimental.pallas import tpu_sc as plsc

_V = 1000
_D = 64
_NPOS = 1024 * 50

_NC = 2
_NS = 16
_L = 16
_NW = _NC * _NS
_BPW = _NPOS // _NW
_CH = 64
_NCH = _BPW // _CH


def _mlse_body(e_ref, w_ref, b_ref, m_ref, lse_ref):
    m = lax.dot_general(
        e_ref[...], w_ref[...], (((1,), (1,)), ((), ())),
        preferred_element_type=jnp.float32,
    )
    m = m + b_ref[...]
    m_ref[...] = m
    mx = jnp.max(m, axis=1, keepdims=True)
    lse_ref[...] = jnp.log(jnp.sum(jnp.exp(m - mx), axis=1, keepdims=True)) + mx


def _sc_body(m_hbm, lse_hbm, tok_hbm, lab_hbm,
             out_hbm, sum_hbm, cnt_hbm,
             tok_v, lab_v, lse_v, rows_v, st_v, sem):
    c = lax.axis_index("c")
    s = lax.axis_index("s")
    wid = s * _NC + c
    base = wid * _BPW
    pltpu.sync_copy(tok_hbm.at[pl.ds(base, _BPW)], tok_v)
    pltpu.sync_copy(lab_hbm.at[pl.ds(base, _BPW)], lab_v)
    pltpu.sync_copy(lse_hbm, lse_v)
    iota16 = lax.iota(jnp.int32, 16)

    def chunk(i, carry):
        nll_acc, cnt_acc = carry
        pltpu.async_copy(m_hbm.at[tok_v.at[pl.ds(i * _CH, _CH)]], rows_v, sem).wait()
        for g in range(_CH // _L):
            off = i * _CH + g * _L
            labs = lab_v[pl.ds(off, _L)]
            toks = tok_v[pl.ds(off, _L)]
            rid = iota16 + g * _L
            vals = plsc.load_gather(rows_v, [rid, labs])
            lses = plsc.load_gather(lse_v, [toks])
            msk = labs != 0
            nll_acc = nll_acc + jnp.where(msk, lses - vals, 0.0)
            cnt_acc = cnt_acc + jnp.where(msk, 1.0, 0.0)
        pltpu.sync_copy(rows_v, out_hbm.at[pl.ds(base + i * _CH, _CH)])
        return nll_acc, cnt_acc

    zero = jnp.zeros((_L,), jnp.float32)
    nll_acc, cnt_acc = lax.fori_loop(0, _NCH, chunk, (zero, zero))
    st_v[0, :] = nll_acc
    st_v[1, :] = cnt_acc
    pltpu.sync_copy(st_v.at[0], sum_hbm.at[wid])
    pltpu.sync_copy(st_v.at[1], cnt_hbm.at[wid])


def _loss_body(s_ref, c_ref, o_ref):
    tot = jnp.sum(s_ref[...])
    cnt = jnp.maximum(jnp.sum(c_ref[...]), 1.0)
    o_ref[...] = jnp.full((1, 1), tot / cnt, jnp.float32)


def kernel(inputs, labels, embed_table, W_out, b_out):
    tok = inputs.reshape(-1).astype(jnp.int32)
    lab = labels.reshape(-1).astype(jnp.int32)

    m, lse2 = pl.pallas_call(
        _mlse_body,
        out_shape=[
            jax.ShapeDtypeStruct((_V, _V), jnp.float32),
            jax.ShapeDtypeStruct((_V, 1), jnp.float32),
        ],
    )(embed_table, W_out, b_out.reshape(1, _V))
    lse = lse2.reshape(_V)

    sc = pl.kernel(
        _sc_body,
        mesh=plsc.VectorSubcoreMesh(core_axis_name="c", subcore_axis_name="s"),
        compiler_params=pltpu.CompilerParams(
            use_tc_tiling_on_sc=False, needs_layout_passes=False),
        out_type=[
            jax.ShapeDtypeStruct((_NPOS, _V), jnp.float32),
            jax.ShapeDtypeStruct((_NW, _L), jnp.float32),
            jax.ShapeDtypeStruct((_NW, _L), jnp.float32),
        ],
        scratch_types=[
            pltpu.VMEM((_BPW,), jnp.int32),
            pltpu.VMEM((_BPW,), jnp.int32),
            pltpu.VMEM((_V,), jnp.float32),
            pltpu.VMEM((_CH, _V), jnp.float32),
            pltpu.VMEM((2, _L), jnp.float32),
            pltpu.SemaphoreType.DMA,
        ],
    )
    logits, sums, cnts = sc(m, lse, tok, lab)

    loss2 = pl.pallas_call(
        _loss_body,
        out_shape=jax.ShapeDtypeStruct((1, 1), jnp.float32),
    )(sums, cnts)
    return loss2.reshape(()), logits


# speedup vs baseline: 2.7903x; 1.4785x over previous
import jax
import jax.numpy as jnp
from jax import lax
from jax.experimental import pallas as pl
from jax.experimental.pallas import tpu as pltpu
from jax.experimental.pallas import tpu_sc as plsc

_V = 1000
_D = 64
_NPOS = 1024 * 50

_NC = 2
_NS = 16
_L = 16
_NW = _NC * _NS
_BPW = _NPOS // _NW
_CG = 80
_NCG = _BPW // _CG

_SC_PARAMS = pltpu.CompilerParams(
    use_tc_tiling_on_sc=False, needs_layout_passes=False)

_ROWS_BLK = 512


def _mlse_body(e_ref, w_ref, b_ref, m_ref, lse_ref):
    m = lax.dot_general(
        e_ref[...], w_ref[...], (((1,), (1,)), ((), ())),
        preferred_element_type=jnp.float32,
    )
    m = m + b_ref[...]
    m_ref[...] = m
    mx = jnp.max(m, axis=1, keepdims=True)
    lse_ref[...] = jnp.log(jnp.sum(jnp.exp(m - mx), axis=1, keepdims=True)) + mx


def _xgather_body(e_hbm, tok_hbm, x_hbm, tok_v, xbuf_v, sem):
    c = lax.axis_index("c")
    s = lax.axis_index("s")
    wid = s * _NC + c
    base = wid * _BPW
    pltpu.sync_copy(tok_hbm.at[pl.ds(base, _BPW)], tok_v)
    cps = [
        pltpu.async_copy(e_hbm.at[tok_v.at[pl.ds(i * _CG, _CG)]],
                         xbuf_v.at[pl.ds(i * _CG, _CG)], sem)
        for i in range(_NCG)
    ]
    for cp in cps:
        cp.wait()
    pltpu.sync_copy(xbuf_v, x_hbm.at[pl.ds(base, _BPW)])


def _nll_body(mflat_hbm, lse_hbm, tok_hbm, lab_hbm, sum_hbm, cnt_hbm,
              tok_v, lab_v, idx2_v, vals_v, lse_v, st_v, sem):
    c = lax.axis_index("c")
    s = lax.axis_index("s")
    wid = s * _NC + c
    base = wid * _BPW
    pltpu.sync_copy(tok_hbm.at[pl.ds(base, _BPW)], tok_v)
    pltpu.sync_copy(lab_hbm.at[pl.ds(base, _BPW)], lab_v)
    pltpu.sync_copy(lse_hbm, lse_v)

    def mk_idx(g, _):
        toks = tok_v[pl.ds(g * _L, _L)]
        labs = lab_v[pl.ds(g * _L, _L)]
        idx2_v[pl.ds(g * _L, _L)] = toks * _V + labs
        return 0

    lax.fori_loop(0, _BPW // _L, mk_idx, 0)
    cps = [
        pltpu.async_copy(mflat_hbm.at[idx2_v.at[pl.ds(i * _CG, _CG)]],
                         vals_v.at[pl.ds(i * _CG, _CG)], sem)
        for i in range(_NCG)
    ]
    for cp in cps:
        cp.wait()

    def acc(g, carry):
        nll_acc, cnt_acc = carry
        labs = lab_v[pl.ds(g * _L, _L)]
        toks = tok_v[pl.ds(g * _L, _L)]
        vals = vals_v[pl.ds(g * _L, _L)]
        lses = plsc.load_gather(lse_v, [toks])
        msk = labs != 0
        nll_acc = nll_acc + jnp.where(msk, lses - vals, 0.0)
        cnt_acc = cnt_acc + jnp.where(msk, 1.0, 0.0)
        return nll_acc, cnt_acc

    zero = jnp.zeros((_L,), jnp.float32)
    nll_acc, cnt_acc = lax.fori_loop(0, _BPW // _L, acc, (zero, zero))
    st_v[0, :] = nll_acc
    st_v[1, :] = cnt_acc
    pltpu.sync_copy(st_v.at[0], sum_hbm.at[wid])
    pltpu.sync_copy(st_v.at[1], cnt_hbm.at[wid])


def _logits_body(x_ref, w_ref, b_ref, o_ref):
    o_ref[...] = lax.dot_general(
        x_ref[...], w_ref[...], (((1,), (1,)), ((), ())),
        preferred_element_type=jnp.float32,
    ) + b_ref[...]


def _loss_body(s_ref, c_ref, o_ref):
    tot = jnp.sum(s_ref[...])
    cnt = jnp.maximum(jnp.sum(c_ref[...]), 1.0)
    o_ref[...] = jnp.full((1, 1), tot / cnt, jnp.float32)


def kernel(inputs, labels, embed_table, W_out, b_out):
    tok = inputs.reshape(-1).astype(jnp.int32)
    lab = labels.reshape(-1).astype(jnp.int32)
    b2 = b_out.reshape(1, _V)

    xg = pl.kernel(
        _xgather_body,
        mesh=plsc.VectorSubcoreMesh(core_axis_name="c", subcore_axis_name="s"),
        compiler_params=_SC_PARAMS,
        out_type=[jax.ShapeDtypeStruct((_NPOS, _D), jnp.float32)],
        scratch_types=[
            pltpu.VMEM((_BPW,), jnp.int32),
            pltpu.VMEM((_BPW, _D), jnp.float32),
            pltpu.SemaphoreType.DMA,
        ],
    )
    (x,) = xg(embed_table, tok)

    m, lse2 = pl.pallas_call(
        _mlse_body,
        out_shape=[
            jax.ShapeDtypeStruct((_V, _V), jnp.float32),
            jax.ShapeDtypeStruct((_V, 1), jnp.float32),
        ],
    )(embed_table, W_out, b2)
    mflat = m.reshape(_V * _V)
    lse = lse2.reshape(_V)

    nll = pl.kernel(
        _nll_body,
        mesh=plsc.VectorSubcoreMesh(core_axis_name="c", subcore_axis_name="s"),
        compiler_params=_SC_PARAMS,
        out_type=[
            jax.ShapeDtypeStruct((_NW, _L), jnp.float32),
            jax.ShapeDtypeStruct((_NW, _L), jnp.float32),
        ],
        scratch_types=[
            pltpu.VMEM((_BPW,), jnp.int32),
            pltpu.VMEM((_BPW,), jnp.int32),
            pltpu.VMEM((_BPW,), jnp.int32),
            pltpu.VMEM((_BPW,), jnp.float32),
            pltpu.VMEM((_V,), jnp.float32),
            pltpu.VMEM((2, _L), jnp.float32),
            pltpu.SemaphoreType.DMA,
        ],
    )
    sums, cnts = nll(mflat, lse, tok, lab)

    logits = pl.pallas_call(
        _logits_body,
        grid=(_NPOS // _ROWS_BLK,),
        in_specs=[
            pl.BlockSpec((_ROWS_BLK, _D), lambda i: (i, 0)),
            pl.BlockSpec((_V, _D), lambda i: (0, 0)),
            pl.BlockSpec((1, _V), lambda i: (0, 0)),
        ],
        out_specs=pl.BlockSpec((_ROWS_BLK, _V), lambda i: (i, 0)),
        out_shape=jax.ShapeDtypeStruct((_NPOS, _V), jnp.float32),
    )(x, W_out, b2)

    loss2 = pl.pallas_call(
        _loss_body,
        out_shape=jax.ShapeDtypeStruct((1, 1), jnp.float32),
    )(sums, cnts)
    return loss2.reshape(()), logits
